# Initial kernel scaffold; baseline (speedup 1.0000x reference)
import functools

import jax
import jax.numpy as jnp
from jax.experimental import pallas as pl
from jax.experimental.pallas import tpu as pltpu
from jax.experimental.pallas import tpu_sc as plsc

_GATHER_WINDOW = 125


def _sc_gather(table, idx_flat):
    B = idx_flat.shape[0]
    D = table.shape[1]
    W = _GATHER_WINDOW
    assert B % W == 0
    idx2 = idx_flat.reshape(1, B)
    mesh = plsc.VectorSubcoreMesh(core_axis_name="c", subcore_axis_name="s")

    @functools.partial(
        pl.kernel,
        out_type=jax.ShapeDtypeStruct((B, D), table.dtype),
        mesh=mesh,
    )
    def gather_kernel(x_hbm, i_hbm, o_hbm):
        def body(i_vmem, o_vmem):
            pltpu.sync_copy(x_hbm.at[i_vmem.at[0]], o_vmem)

        pltpu.emit_pipeline(
            body,
            grid=(B // W,),
            in_specs=[pl.BlockSpec((1, W), lambda i: (0, i))],
            out_specs=[pl.BlockSpec((W, D), lambda i: (i, 0))],
            core_axis_name=("c", "s"),
            dimension_semantics=(pltpu.PARALLEL,),
        )(i_hbm, o_hbm)

    return gather_kernel(table, idx2)


_EDGE_BLK = 2560


def _mlp_body(src_ref, dst_ref, bond_ref, w1a_ref, w1b_ref, b1_ref,
              w2_ref, b2_ref, w3_ref, b3_ref, gamma_ref, beta_ref, out_ref):
    m = src_ref[0] + dst_ref[0]
    h = (jnp.dot(bond_ref[...], w1a_ref[...], preferred_element_type=jnp.float32)
         + jnp.dot(m, w1b_ref[...], preferred_element_type=jnp.float32)
         + b1_ref[...])
    h = jnp.where(h > 0, h, 0.01 * h)
    h = jnp.dot(h, w2_ref[...], preferred_element_type=jnp.float32) + b2_ref[...]
    h = jnp.where(h > 0, h, 0.01 * h)
    h = jnp.dot(h, w3_ref[...], preferred_element_type=jnp.float32) + b3_ref[...]
    h = jnp.where(h > 0, h, 0.01 * h)
    scale = gamma_ref[...] * jax.lax.rsqrt(jnp.float32(1.0 + 1e-5))
    out_ref[...] = scale * h + beta_ref[...]


def _tc_mlp(gathered, bond_feats, W1, b1, W2, b2, W3, b3, gamma, beta):
    E, D_EDGE = bond_feats.shape
    D = W1.shape[1]
    g3 = gathered.reshape(2, E, D)
    blk = _EDGE_BLK
    assert E % blk == 0
    w1a = W1[:D_EDGE]
    w1b = W1[D_EDGE:]
    row = lambda v: v.reshape(1, D)

    return pl.pallas_call(
        _mlp_body,
        grid=(E // blk,),
        in_specs=[
            pl.BlockSpec((1, blk, D), lambda i: (0, i, 0)),
            pl.BlockSpec((1, blk, D), lambda i: (1, i, 0)),
            pl.BlockSpec((blk, D_EDGE), lambda i: (i, 0)),
            pl.BlockSpec((D_EDGE, D), lambda i: (0, 0)),
            pl.BlockSpec((D, D), lambda i: (0, 0)),
            pl.BlockSpec((1, D), lambda i: (0, 0)),
            pl.BlockSpec((D, D), lambda i: (0, 0)),
            pl.BlockSpec((1, D), lambda i: (0, 0)),
            pl.BlockSpec((D, D), lambda i: (0, 0)),
            pl.BlockSpec((1, D), lambda i: (0, 0)),
            pl.BlockSpec((1, D), lambda i: (0, 0)),
            pl.BlockSpec((1, D), lambda i: (0, 0)),
        ],
        out_specs=pl.BlockSpec((blk, D), lambda i: (i, 0)),
        out_shape=jax.ShapeDtypeStruct((E, D), jnp.float32),
    )(g3, g3, bond_feats, w1a, w1b, row(b1), W2, row(b2), W3, row(b3),
      row(gamma), row(beta))


def kernel(atom_feats, edge_index, bond_feats, W1, b1, W2, b2, W3, b3,
           gamma, beta):
    idx_flat = edge_index.reshape(-1)
    gathered = _sc_gather(atom_feats, idx_flat)
    return _tc_mlp(gathered, bond_feats, W1, b1, W2, b2, W3, b3, gamma, beta)


# speedup vs baseline: 2.9511x; 2.9511x over previous
import functools

import jax
import jax.numpy as jnp
from jax.experimental import pallas as pl
from jax.experimental.pallas import tpu as pltpu
from jax.experimental.pallas import tpu_sc as plsc

_GATHER_WINDOW = 128


def _sc_gather(table, idx_flat):
    B = idx_flat.shape[0]
    D = table.shape[1]
    W = _GATHER_WINDOW
    assert B % W == 0
    idx2 = idx_flat.reshape(1, B)
    mesh = plsc.VectorSubcoreMesh(core_axis_name="c", subcore_axis_name="s")

    @functools.partial(
        pl.kernel,
        out_type=jax.ShapeDtypeStruct((B, D), table.dtype),
        mesh=mesh,
    )
    def gather_kernel(x_hbm, i_hbm, o_hbm):
        def body(i_vmem, o_vmem):
            pltpu.sync_copy(x_hbm.at[i_vmem.at[0]], o_vmem)

        pltpu.emit_pipeline(
            body,
            grid=(B // W,),
            in_specs=[pl.BlockSpec((1, W), lambda i: (0, i))],
            out_specs=[pl.BlockSpec((W, D), lambda i: (i, 0))],
            core_axis_name=("c", "s"),
            dimension_semantics=(pltpu.PARALLEL,),
        )(i_hbm, o_hbm)

    return gather_kernel(table, idx2)


_EDGE_BLK = 2560


def _mlp_body(src_ref, dst_ref, bond_ref, w1a_ref, w1b_ref, b1_ref,
              w2_ref, b2_ref, w3_ref, b3_ref, gamma_ref, beta_ref, out_ref):
    m = src_ref[0] + dst_ref[0]
    h = (jnp.dot(bond_ref[...], w1a_ref[...], preferred_element_type=jnp.float32)
         + jnp.dot(m, w1b_ref[...], preferred_element_type=jnp.float32)
         + b1_ref[...])
    h = jnp.where(h > 0, h, 0.01 * h)
    h = jnp.dot(h, w2_ref[...], preferred_element_type=jnp.float32) + b2_ref[...]
    h = jnp.where(h > 0, h, 0.01 * h)
    h = jnp.dot(h, w3_ref[...], preferred_element_type=jnp.float32) + b3_ref[...]
    h = jnp.where(h > 0, h, 0.01 * h)
    scale = gamma_ref[...] * jax.lax.rsqrt(jnp.float32(1.0 + 1e-5))
    out_ref[...] = scale * h + beta_ref[...]


def _tc_mlp(gathered, bond_feats, W1, b1, W2, b2, W3, b3, gamma, beta):
    E, D_EDGE = bond_feats.shape
    D = W1.shape[1]
    g3 = gathered.reshape(2, E, D)
    blk = _EDGE_BLK
    assert E % blk == 0
    w1a = W1[:D_EDGE]
    w1b = W1[D_EDGE:]
    row = lambda v: v.reshape(1, D)

    return pl.pallas_call(
        _mlp_body,
        grid=(E // blk,),
        in_specs=[
            pl.BlockSpec((1, blk, D), lambda i: (0, i, 0)),
            pl.BlockSpec((1, blk, D), lambda i: (1, i, 0)),
            pl.BlockSpec((blk, D_EDGE), lambda i: (i, 0)),
            pl.BlockSpec((D_EDGE, D), lambda i: (0, 0)),
            pl.BlockSpec((D, D), lambda i: (0, 0)),
            pl.BlockSpec((1, D), lambda i: (0, 0)),
            pl.BlockSpec((D, D), lambda i: (0, 0)),
            pl.BlockSpec((1, D), lambda i: (0, 0)),
            pl.BlockSpec((D, D), lambda i: (0, 0)),
            pl.BlockSpec((1, D), lambda i: (0, 0)),
            pl.BlockSpec((1, D), lambda i: (0, 0)),
            pl.BlockSpec((1, D), lambda i: (0, 0)),
        ],
        out_specs=pl.BlockSpec((blk, D), lambda i: (i, 0)),
        out_shape=jax.ShapeDtypeStruct((E, D), jnp.float32),
    )(g3, g3, bond_feats, w1a, w1b, row(b1), W2, row(b2), W3, row(b3),
      row(gamma), row(beta))


def kernel(atom_feats, edge_index, bond_feats, W1, b1, W2, b2, W3, b3,
           gamma, beta):
    idx_flat = edge_index.reshape(-1)
    gathered = _sc_gather(atom_feats, idx_flat)
    return _tc_mlp(gathered, bond_feats, W1, b1, W2, b2, W3, b3, gamma, beta)


# speedup vs baseline: 3.2253x; 1.0929x over previous
import functools

import jax
import jax.numpy as jnp
from jax.experimental import pallas as pl
from jax.experimental.pallas import tpu as pltpu
from jax.experimental.pallas import tpu_sc as plsc

_GATHER_WINDOW = 128
_EDGE_BLK = 2560
_N_CHUNKS = 5


def _sc_gather(table, idx_flat):
    B = idx_flat.shape[0]
    D = table.shape[1]
    W = _GATHER_WINDOW
    assert B % W == 0
    idx2 = idx_flat.reshape(1, B)
    mesh = plsc.VectorSubcoreMesh(core_axis_name="c", subcore_axis_name="s")

    @functools.partial(
        pl.kernel,
        out_type=jax.ShapeDtypeStruct((B, D), table.dtype),
        mesh=mesh,
    )
    def gather_kernel(x_hbm, i_hbm, o_hbm):
        def body(i_vmem, o_vmem):
            pltpu.sync_copy(x_hbm.at[i_vmem.at[0]], o_vmem)

        pltpu.emit_pipeline(
            body,
            grid=(B // W,),
            in_specs=[pl.BlockSpec((1, W), lambda i: (0, i))],
            out_specs=[pl.BlockSpec((W, D), lambda i: (i, 0))],
            core_axis_name=("c", "s"),
            dimension_semantics=(pltpu.PARALLEL,),
        )(i_hbm, o_hbm)

    return gather_kernel(table, idx2)


def _mlp_body(src_ref, dst_ref, bond_ref, w1a_ref, w1b_ref, b1_ref,
              w2_ref, b2_ref, w3_ref, b3_ref, gamma_ref, beta_ref, out_ref):
    m = src_ref[0] + dst_ref[0]
    h = (jnp.dot(bond_ref[...], w1a_ref[...], preferred_element_type=jnp.float32)
         + jnp.dot(m, w1b_ref[...], preferred_element_type=jnp.float32)
         + b1_ref[...])
    h = jnp.where(h > 0, h, 0.01 * h)
    h = jnp.dot(h, w2_ref[...], preferred_element_type=jnp.float32) + b2_ref[...]
    h = jnp.where(h > 0, h, 0.01 * h)
    h = jnp.dot(h, w3_ref[...], preferred_element_type=jnp.float32) + b3_ref[...]
    h = jnp.where(h > 0, h, 0.01 * h)
    scale = gamma_ref[...] * jax.lax.rsqrt(jnp.float32(1.0 + 1e-5))
    out_ref[...] = scale * h + beta_ref[...]


def _tc_mlp_chunk(prev_buf, gathered, bond_feats, w1a, w1b, b1, W2, b2,
                  W3, b3, gamma, beta, base_blk, n_blk):
    E, D_EDGE = bond_feats.shape
    D = W2.shape[1]
    Ec = gathered.shape[0] // 2
    g3 = gathered.reshape(2, Ec, D)
    blk = _EDGE_BLK
    full = lambda r, c: pl.BlockSpec((r, c), lambda i: (0, 0))

    in_specs = [
        pl.BlockSpec((1, blk, D), lambda i: (0, i, 0)),
        pl.BlockSpec((1, blk, D), lambda i: (1, i, 0)),
        pl.BlockSpec((blk, D_EDGE), lambda i: (base_blk + i, 0)),
        full(D_EDGE, D),
        full(D, D),
        full(1, D),
        full(D, D),
        full(1, D),
        full(D, D),
        full(1, D),
        full(1, D),
        full(1, D),
    ]
    args = [g3, g3, bond_feats, w1a, w1b, b1, W2, b2, W3, b3, gamma, beta]
    body = _mlp_body
    aliases = {}
    if prev_buf is not None:
        in_specs = [pl.BlockSpec(memory_space=pl.ANY)] + in_specs
        args = [prev_buf] + args
        aliases = {0: 0}
        body = lambda prev_ref, *rest: _mlp_body(*rest)

    return pl.pallas_call(
        body,
        grid=(n_blk,),
        in_specs=in_specs,
        out_specs=pl.BlockSpec((blk, D), lambda i: (base_blk + i, 0)),
        out_shape=jax.ShapeDtypeStruct((E, D), jnp.float32),
        input_output_aliases=aliases,
    )(*args)


def kernel(atom_feats, edge_index, bond_feats, W1, b1, W2, b2, W3, b3,
           gamma, beta):
    E, D_EDGE = bond_feats.shape
    D = W1.shape[1]
    w1a = W1[:D_EDGE]
    w1b = W1[D_EDGE:]
    row = lambda v: v.reshape(1, D)
    b1r, b2r, b3r = row(b1), row(b2), row(b3)
    gr, br = row(gamma), row(beta)

    n_chunks = _N_CHUNKS
    Ec = E // n_chunks
    assert Ec * n_chunks == E and Ec % _EDGE_BLK == 0
    n_blk = Ec // _EDGE_BLK

    gathered = [
        _sc_gather(atom_feats,
                   edge_index[:, k * Ec:(k + 1) * Ec].reshape(-1))
        for k in range(n_chunks)
    ]

    buf = None
    for k in range(n_chunks):
        buf = _tc_mlp_chunk(buf, gathered[k], bond_feats,
                            w1a, w1b, b1r, W2, b2r, W3, b3r, gr, br,
                            base_blk=k * n_blk, n_blk=n_blk)
    return buf
